# baseline (speedup 1.0000x reference)
import jax
import jax.numpy as jnp
from jax import lax
from jax.experimental import pallas as pl
from jax.experimental.pallas import tpu as pltpu


def _round_up(x, m):
    return (x + m - 1) // m * m


def _logits_kernel(x_ref, emb_ref, bias_ref, out_ref):
    xb = x_ref[...].astype(jnp.bfloat16)
    eb = emb_ref[...].astype(jnp.bfloat16)
    acc = lax.dot_general(
        xb, eb,
        dimension_numbers=(((1,), (1,)), ((), ())),
        preferred_element_type=jnp.float32)
    out_ref[...] = acc + bias_ref[...]


def kernel(x, emb_weight, bias):
    B, D = x.shape
    V = emb_weight.shape[0]

    tv = 1920
    V_pad = _round_up(V, tv)
    nv = V_pad // tv

    B_p = _round_up(B, 8)
    x_p = x if B_p == B else jnp.pad(x, ((0, B_p - B), (0, 0)))
    bias_p = bias.astype(jnp.float32)
    if V_pad != V:
        bias_p = jnp.pad(bias_p, ((0, 0), (0, V_pad - V)))

    out = pl.pallas_call(
        _logits_kernel,
        out_shape=jax.ShapeDtypeStruct((B_p, V_pad), jnp.float32),
        grid=(nv,),
        in_specs=[
            pl.BlockSpec((B_p, D), lambda v: (0, 0)),
            pl.BlockSpec((tv, D), lambda v: (v, 0)),
            pl.BlockSpec((1, tv), lambda v: (0, v)),
        ],
        out_specs=pl.BlockSpec((B_p, tv), lambda v: (0, v)),
        compiler_params=pltpu.CompilerParams(
            dimension_semantics=("parallel",),
            vmem_limit_bytes=64 * 1024 * 1024,
        ),
    )(x_p, emb_weight, bias_p)
    if B_p != B or V_pad != V:
        out = out[:B, :V]
    return out


# speedup vs baseline: 1.0127x; 1.0127x over previous
import jax
import jax.numpy as jnp
from jax import lax
from jax.experimental import pallas as pl
from jax.experimental.pallas import tpu as pltpu


def _round_up(x, m):
    return (x + m - 1) // m * m


def _logits_kernel(x_ref, emb_ref, bias_ref, out_ref):
    xb = x_ref[...].astype(jnp.bfloat16)
    eb = emb_ref[...].astype(jnp.bfloat16)
    acc = lax.dot_general(
        xb, eb,
        dimension_numbers=(((1,), (1,)), ((), ())),
        preferred_element_type=jnp.float32)
    out_ref[...] = acc + bias_ref[...]


def kernel(x, emb_weight, bias):
    B, D = x.shape
    V = emb_weight.shape[0]

    tv = 1024
    V_pad = _round_up(V, tv)
    nv = V_pad // tv

    B_p = _round_up(B, 8)
    x_p = x if B_p == B else jnp.pad(x, ((0, B_p - B), (0, 0)))
    bias_p = bias.astype(jnp.float32)
    if V_pad != V:
        bias_p = jnp.pad(bias_p, ((0, 0), (0, V_pad - V)))

    out = pl.pallas_call(
        _logits_kernel,
        out_shape=jax.ShapeDtypeStruct((B_p, V_pad), jnp.float32),
        grid=(nv,),
        in_specs=[
            pl.BlockSpec((B_p, D), lambda v: (0, 0)),
            pl.BlockSpec((tv, D), lambda v: (v, 0)),
            pl.BlockSpec((1, tv), lambda v: (0, v)),
        ],
        out_specs=pl.BlockSpec((B_p, tv), lambda v: (0, v)),
        compiler_params=pltpu.CompilerParams(
            dimension_semantics=("parallel",),
            vmem_limit_bytes=64 * 1024 * 1024,
        ),
    )(x_p, emb_weight, bias_p)
    if B_p != B or V_pad != V:
        out = out[:B, :V]
    return out


# speedup vs baseline: 2.0967x; 2.0704x over previous
import jax
import jax.numpy as jnp
from jax import lax
from jax.experimental import pallas as pl
from jax.experimental.pallas import tpu as pltpu


def _round_up(x, m):
    return (x + m - 1) // m * m


def _logits_kernel(x_ref, emb_ref, bias_ref, out_ref):
    xb = x_ref[...].astype(jnp.bfloat16)
    eb = emb_ref[...].astype(jnp.bfloat16).T
    acc = lax.dot_general(
        xb, eb,
        dimension_numbers=(((1,), (0,)), ((), ())),
        preferred_element_type=jnp.float32)
    out_ref[...] = acc + bias_ref[...]


def kernel(x, emb_weight, bias):
    B, D = x.shape
    V = emb_weight.shape[0]

    tv = next((t for t in (1280, 640, 512, 768, 384, 256, 128) if V % t == 0), 512)
    V_pad = _round_up(V, tv)
    nv = V_pad // tv

    B_p = _round_up(B, 8)
    x_p = x if B_p == B else jnp.pad(x, ((0, B_p - B), (0, 0)))
    bias_p = bias.astype(jnp.float32)
    if V_pad != V:
        bias_p = jnp.pad(bias_p, ((0, 0), (0, V_pad - V)))

    out = pl.pallas_call(
        _logits_kernel,
        out_shape=jax.ShapeDtypeStruct((B_p, V_pad), jnp.float32),
        grid=(nv,),
        in_specs=[
            pl.BlockSpec((B_p, D), lambda v: (0, 0)),
            pl.BlockSpec((tv, D), lambda v: (v, 0)),
            pl.BlockSpec((1, tv), lambda v: (0, v)),
        ],
        out_specs=pl.BlockSpec((B_p, tv), lambda v: (0, v)),
        compiler_params=pltpu.CompilerParams(
            dimension_semantics=("parallel",),
            vmem_limit_bytes=64 * 1024 * 1024,
        ),
    )(x_p, emb_weight, bias_p)
    if B_p != B or V_pad != V:
        out = out[:B, :V]
    return out
